# baseline (device time: 109746 ns/iter reference)
import jax
import jax.numpy as jnp
from jax import lax
from jax.experimental import pallas as pl
from jax.experimental.pallas import tpu as pltpu

N_DEV = 4
E_PER_DEV = 4
N_EXPERTS = N_DEV * E_PER_DEV


def kernel(x, router_W, route_idx, expert_W, shared_W):
    n_tok, d_model = x.shape
    e_loc, _, d_ff = expert_W.shape

    ew_b = expert_W.astype(jnp.bfloat16)
    sw_b = shared_W.astype(jnp.bfloat16)

    def body(x_ref, rw_ref, idx_ref, ew_ref, sw_ref, out_ref,
             commR, commL, sR, rR, sL, rL):
        my = lax.axis_index("i")
        left = lax.rem(my + N_DEV - 1, N_DEV)
        right = lax.rem(my + 1, N_DEV)
        opp = lax.rem(my + 2, N_DEV)

        barrier_sem = pltpu.get_barrier_semaphore()
        for nbr in (left, right):
            pl.semaphore_signal(barrier_sem, inc=1, device_id=(nbr,),
                                device_id_type=pl.DeviceIdType.MESH)
        pl.semaphore_wait(barrier_sem, 2)

        def mk(src, dst, ssem, rsem, dev):
            return pltpu.make_async_remote_copy(
                src_ref=src, dst_ref=dst, send_sem=ssem, recv_sem=rsem,
                device_id=(dev,), device_id_type=pl.DeviceIdType.MESH)

        toR = [mk(ew_ref.at[k], commR.at[k], sR.at[k], rR.at[k], right)
               for k in range(4)]
        toR.append(mk(commR.at[0], commR.at[4], sR.at[4], rR.at[4], right))
        toR.append(mk(commR.at[1], commR.at[5], sR.at[5], rR.at[5], right))

        jL = (2, 3, 0, 1)
        toL = [mk(ew_ref.at[jL[m]], commL.at[m], sL.at[m], rL.at[m], left)
               for m in range(4)]
        toL.append(mk(commL.at[0], commL.at[4], sL.at[4], rL.at[4], left))
        toL.append(mk(commL.at[1], commL.at[5], sL.at[5], rL.at[5], left))

        for k in range(4):
            toR[k].start()
        for m in range(4):
            toL[m].start()

        x32 = x_ref[:, :]
        scores = jnp.dot(x32, rw_ref[:, :], preferred_element_type=jnp.float32)
        m = jnp.max(scores, axis=-1, keepdims=True)
        ex = jnp.exp(scores - m)
        probs = ex / jnp.sum(ex, axis=-1, keepdims=True)
        idx = idx_ref[:, :]
        onehot = lax.broadcasted_iota(jnp.int32, (n_tok, N_EXPERTS), 1) == idx
        p_top = jnp.sum(jnp.where(onehot, probs, 0.0), axis=-1, keepdims=True)

        xb = x32.astype(jnp.bfloat16)

        out_ref[:, :] = jnp.dot(xb, sw_ref[:, :],
                                preferred_element_type=jnp.float32)

        def acc_group(items):
            cs = [jnp.dot(xb, r[:, :], preferred_element_type=jnp.float32)
                  for r, _ in items]
            sel = cs[-1]
            for (_, e_g), c in list(zip(items, cs))[-2::-1]:
                sel = jnp.where(idx == e_g, c, sel)
            gm = (idx == items[0][1])
            for _, e_g in items[1:]:
                gm = gm | (idx == e_g)
            out_ref[:, :] += jnp.where(gm, p_top * sel, jnp.float32(0.0))

        acc_group([(ew_ref.at[j], my * 4 + j) for j in range(4)])

        toR[0].wait_recv(); toR[4].start()
        toL[0].wait_recv(); toL[4].start()
        toR[1].wait_recv(); toR[5].start()
        toL[1].wait_recv(); toL[5].start()
        acc_group([(commR.at[0], left * 4 + 0), (commL.at[0], right * 4 + 2),
                   (commR.at[1], left * 4 + 1), (commL.at[1], right * 4 + 3)])

        toR[2].wait_recv(); toR[3].wait_recv()
        toL[2].wait_recv(); toL[3].wait_recv()
        acc_group([(commR.at[2], left * 4 + 2), (commR.at[3], left * 4 + 3),
                   (commL.at[2], right * 4 + 0), (commL.at[3], right * 4 + 1)])

        toR[4].wait_recv(); toR[5].wait_recv()
        toL[4].wait_recv(); toL[5].wait_recv()
        acc_group([(commR.at[4], opp * 4 + 0), (commR.at[5], opp * 4 + 1),
                   (commL.at[4], opp * 4 + 2), (commL.at[5], opp * 4 + 3)])

        for d in toR + toL:
            d.wait_send()

    return pl.pallas_call(
        body,
        out_shape=jax.ShapeDtypeStruct((n_tok, d_ff), jnp.float32),
        in_specs=[pl.BlockSpec(memory_space=pltpu.VMEM)] * 5,
        out_specs=pl.BlockSpec(memory_space=pltpu.VMEM),
        scratch_shapes=[
            pltpu.VMEM((6, d_model, d_ff), jnp.bfloat16),
            pltpu.VMEM((6, d_model, d_ff), jnp.bfloat16),
            pltpu.SemaphoreType.DMA((6,)),
            pltpu.SemaphoreType.DMA((6,)),
            pltpu.SemaphoreType.DMA((6,)),
            pltpu.SemaphoreType.DMA((6,)),
        ],
        compiler_params=pltpu.CompilerParams(
            collective_id=0, vmem_limit_bytes=100 * 1024 * 1024),
    )(x, router_W, route_idx, ew_b, sw_b)


# device time: 91186 ns/iter; 1.2035x vs baseline; 1.2035x over previous
import jax
import jax.numpy as jnp
from jax import lax
from jax.experimental import pallas as pl
from jax.experimental.pallas import tpu as pltpu

N_DEV = 4
E_PER_DEV = 4
N_EXPERTS = N_DEV * E_PER_DEV


def kernel(x, router_W, route_idx, expert_W, shared_W):
    n_tok, d_model = x.shape
    e_loc, _, d_ff = expert_W.shape

    ew_b = expert_W.astype(jnp.bfloat16)
    sw_b = shared_W.astype(jnp.bfloat16)

    def body(x_ref, rw_ref, idx_ref, ew_ref, sw_ref, out_ref,
             commR, commL, sR, rR, sL, rL):
        my = lax.axis_index("i")
        left = lax.rem(my + N_DEV - 1, N_DEV)
        right = lax.rem(my + 1, N_DEV)
        opp = lax.rem(my + 2, N_DEV)

        barrier_sem = pltpu.get_barrier_semaphore()
        for nbr in (left, right):
            pl.semaphore_signal(barrier_sem, inc=1, device_id=(nbr,),
                                device_id_type=pl.DeviceIdType.MESH)
        pl.semaphore_wait(barrier_sem, 2)

        def mk(src, dst, ssem, rsem, dev):
            return pltpu.make_async_remote_copy(
                src_ref=src, dst_ref=dst, send_sem=ssem, recv_sem=rsem,
                device_id=(dev,), device_id_type=pl.DeviceIdType.MESH)

        toR = [mk(ew_ref.at[k], commR.at[k], sR.at[k], rR.at[k], right)
               for k in range(4)]
        toR.append(mk(commR.at[0], commR.at[4], sR.at[4], rR.at[4], right))
        toR.append(mk(commR.at[1], commR.at[5], sR.at[5], rR.at[5], right))

        jL = (2, 3, 0, 1)
        toL = [mk(ew_ref.at[jL[m]], commL.at[m], sL.at[m], rL.at[m], left)
               for m in range(4)]
        toL.append(mk(commL.at[0], commL.at[4], sL.at[4], rL.at[4], left))
        toL.append(mk(commL.at[1], commL.at[5], sL.at[5], rL.at[5], left))

        for k in range(4):
            toR[k].start()
        for m in range(4):
            toL[m].start()

        x32 = x_ref[:, :]
        scores = jnp.dot(x32, rw_ref[:, :], preferred_element_type=jnp.float32)
        m = jnp.max(scores, axis=-1, keepdims=True)
        ex = jnp.exp(scores - m)
        probs = ex / jnp.sum(ex, axis=-1, keepdims=True)
        idx = idx_ref[:, :]
        onehot = lax.broadcasted_iota(jnp.int32, (n_tok, N_EXPERTS), 1) == idx
        p_top = jnp.sum(jnp.where(onehot, probs, 0.0), axis=-1, keepdims=True)

        xb = x32.astype(jnp.bfloat16)

        out_ref[:, :] = jnp.dot(xb, sw_ref[:, :],
                                preferred_element_type=jnp.float32)

        def acc(block_ref, e_g):
            contrib = jnp.dot(xb, block_ref[:, :],
                              preferred_element_type=jnp.float32)
            p_e = jnp.where(idx == e_g, p_top, jnp.float32(0.0))
            out_ref[:, :] += p_e * contrib

        toR[0].wait_recv(); toR[4].start(); acc(commR.at[0], left * 4 + 0)
        toL[0].wait_recv(); toL[4].start(); acc(commL.at[0], right * 4 + 2)
        toR[1].wait_recv(); toR[5].start(); acc(commR.at[1], left * 4 + 1)
        toL[1].wait_recv(); toL[5].start(); acc(commL.at[1], right * 4 + 3)

        for j in range(4):
            acc(ew_ref.at[j], my * 4 + j)

        toR[2].wait_recv(); acc(commR.at[2], left * 4 + 2)
        toR[3].wait_recv(); acc(commR.at[3], left * 4 + 3)
        toL[2].wait_recv(); acc(commL.at[2], right * 4 + 0)
        toL[3].wait_recv(); acc(commL.at[3], right * 4 + 1)

        toR[4].wait_recv(); acc(commR.at[4], opp * 4 + 0)
        toR[5].wait_recv(); acc(commR.at[5], opp * 4 + 1)
        toL[4].wait_recv(); acc(commL.at[4], opp * 4 + 2)
        toL[5].wait_recv(); acc(commL.at[5], opp * 4 + 3)

        for d in toR + toL:
            d.wait_send()

    return pl.pallas_call(
        body,
        out_shape=jax.ShapeDtypeStruct((n_tok, d_ff), jnp.float32),
        in_specs=[pl.BlockSpec(memory_space=pltpu.VMEM)] * 5,
        out_specs=pl.BlockSpec(memory_space=pltpu.VMEM),
        scratch_shapes=[
            pltpu.VMEM((6, d_model, d_ff), jnp.bfloat16),
            pltpu.VMEM((6, d_model, d_ff), jnp.bfloat16),
            pltpu.SemaphoreType.DMA((6,)),
            pltpu.SemaphoreType.DMA((6,)),
            pltpu.SemaphoreType.DMA((6,)),
            pltpu.SemaphoreType.DMA((6,)),
        ],
        compiler_params=pltpu.CompilerParams(collective_id=0),
    )(x, router_W, route_idx, ew_b, sw_b)
